# baseline (speedup 1.0000x reference)
import jax
import jax.numpy as jnp
from jax import lax
from jax.experimental import pallas as pl
from jax.experimental.pallas import tpu as pltpu
from jax.experimental.pallas import tpu_sc as plsc

N = 10000
E = 320000
D = 128
DE = 16
OUT = 128

NC = 2
NS = 16
NW = NC * NS
RPW = 312
LAST_ROWS = N - (NW - 1) * RPW
CHUNK = 2000
NCHUNKS = E // CHUNK
G = 128
LCAP = CHUNK + G


def _sc_agg_body(src_hbm, dst_hbm, p1_hbm, p2_hbm, e3_hbm, agg_hbm,
                 dstv, srcv, ids_buf, srcc, dstc, gsrc, gdst, gids,
                 p1b, p2b, e3b, acc, sem_lin, sem_g):
    cid = lax.axis_index("c")
    sid = lax.axis_index("s")
    wid = sid * NC + cid
    n0 = wid * RPW
    n1 = jnp.where(wid == NW - 1, N, n0 + RPW)
    lanes = lax.iota(jnp.int32, 16)

    zf = jnp.zeros((16,), jnp.float32)

    def _zacc(i, carry):
        acc[i // 8, pl.ds((i % 8) * 16, 16)] = zf
        return carry

    lax.fori_loop(0, LAST_ROWS * 8, _zacc, 0)

    zi = jnp.zeros((16,), jnp.int32)

    def _zidx(i, carry):
        s = pl.ds(i * 16, 16)
        ids_buf[s] = zi
        srcc[s] = zi
        dstc[s] = zi
        return carry

    lax.fori_loop(0, LCAP // 16, _zidx, 0)

    def scan_chunk(c):
        e0 = c * CHUNK

        def scan_body(v, m):
            s = pl.ds(v * 16, 16)
            dv = dstv[s]
            sv = srcv[s]
            mask = (dv >= n0) & (dv < n1)
            ids = e0 + v * 16 + lanes
            cum = plsc.cumsum(mask.astype(jnp.int32))
            pos = m + cum - 1
            plsc.store_scatter(ids_buf, [pos], ids, mask=mask)
            plsc.store_scatter(srcc, [pos], sv, mask=mask)
            plsc.store_scatter(dstc, [pos], dv, mask=mask)
            return m + cum[15]

        return lax.fori_loop(0, CHUNK // 16, scan_body, jnp.int32(0))

    def stage_block(off):
        for k in range(G // 16):
            si = pl.ds(off + k * 16, 16)
            di = pl.ds(k * 16, 16)
            gsrc[di] = srcc[si]
            gdst[di] = dstc[si]
            gids[di] = ids_buf[si]

    def fire_gather():
        pltpu.async_copy(p1_hbm.at[gsrc], p1b, sem_g)
        pltpu.async_copy(p2_hbm.at[gdst], p2b, sem_g)
        pltpu.async_copy(e3_hbm.at[gids], e3b, sem_g)

    def wait_gather():
        pltpu.make_async_copy(p1_hbm.at[pl.ds(0, G)], p1b, sem_g).wait()
        pltpu.make_async_copy(p1_hbm.at[pl.ds(0, G)], p2b, sem_g).wait()
        pltpu.make_async_copy(p1_hbm.at[pl.ds(0, G)], e3b, sem_g).wait()

    def accumulate(m_eff):
        def acc_body(j, carry):
            base16 = (j // 16) * 16
            dvec = gdst[pl.ds(base16, 16)]
            lane = j - base16
            d = jnp.sum(jnp.where(lanes == lane, dvec, 0)) - n0
            for k in range(OUT // 16):
                sk = pl.ds(k * 16, 16)
                z = p1b[j, sk] + p2b[j, sk] + e3b[j, sk]
                acc[d, sk] = jnp.maximum(acc[d, sk], z)
            return carry

        lax.fori_loop(0, m_eff, acc_body, 0)

    def accumulate_extra(m_tot):
        nex = (m_tot + (G - 1)) // G - 1

        def ov(b, carry):
            off = (b + 1) * G
            stage_block(off)
            fire_gather()
            wait_gather()
            accumulate(jnp.minimum(m_tot - off, G))
            return carry

        lax.fori_loop(0, nex, ov, 0)

    pltpu.sync_copy(dst_hbm.at[pl.ds(0, CHUNK)], dstv)
    pltpu.sync_copy(src_hbm.at[pl.ds(0, CHUNK)], srcv)
    m0 = scan_chunk(0)
    stage_block(0)
    fire_gather()
    nxt0 = jnp.minimum(CHUNK, E - CHUNK)
    pltpu.async_copy(dst_hbm.at[pl.ds(nxt0, CHUNK)], dstv, sem_lin)
    pltpu.async_copy(src_hbm.at[pl.ds(nxt0, CHUNK)], srcv, sem_lin)

    def body(i, m_prev):
        early = m_prev > G

        @pl.when(early)
        def _():
            wait_gather()
            accumulate(G)
            accumulate_extra(m_prev)

        pltpu.make_async_copy(dst_hbm.at[pl.ds(0, CHUNK)], dstv,
                              sem_lin).wait()
        pltpu.make_async_copy(src_hbm.at[pl.ds(0, CHUNK)], srcv,
                              sem_lin).wait()
        m_i = scan_chunk(i)

        @pl.when(jnp.logical_not(early))
        def _():
            wait_gather()
            accumulate(jnp.minimum(m_prev, G))

        stage_block(0)
        fire_gather()
        nxt = jnp.minimum((i + 1) * CHUNK, E - CHUNK)
        pltpu.async_copy(dst_hbm.at[pl.ds(nxt, CHUNK)], dstv, sem_lin)
        pltpu.async_copy(src_hbm.at[pl.ds(nxt, CHUNK)], srcv, sem_lin)
        return m_i

    m_last = lax.fori_loop(1, NCHUNKS, body, m0)

    pltpu.make_async_copy(dst_hbm.at[pl.ds(0, CHUNK)], dstv, sem_lin).wait()
    pltpu.make_async_copy(src_hbm.at[pl.ds(0, CHUNK)], srcv, sem_lin).wait()
    wait_gather()
    accumulate(jnp.minimum(m_last, G))
    accumulate_extra(m_last)

    @pl.when(wid < NW - 1)
    def _():
        pltpu.sync_copy(acc.at[:RPW], agg_hbm.at[pl.ds(n0, RPW)])

    @pl.when(wid == NW - 1)
    def _():
        pltpu.sync_copy(acc, agg_hbm.at[pl.ds(n0, LAST_ROWS)])


def _sc_agg(src, dst, P1, P2, E3):
    mesh = plsc.VectorSubcoreMesh(core_axis_name="c", subcore_axis_name="s")
    return pl.kernel(
        _sc_agg_body,
        out_type=jax.ShapeDtypeStruct((N, OUT), jnp.float32),
        mesh=mesh,
        compiler_params=pltpu.CompilerParams(needs_layout_passes=False),
        scratch_types=[
            pltpu.VMEM((CHUNK,), jnp.int32),
            pltpu.VMEM((CHUNK,), jnp.int32),
            pltpu.VMEM((LCAP,), jnp.int32),
            pltpu.VMEM((LCAP,), jnp.int32),
            pltpu.VMEM((LCAP,), jnp.int32),
            pltpu.VMEM((G,), jnp.int32),
            pltpu.VMEM((G,), jnp.int32),
            pltpu.VMEM((G,), jnp.int32),
            pltpu.VMEM((G, OUT), jnp.float32),
            pltpu.VMEM((G, OUT), jnp.float32),
            pltpu.VMEM((G, OUT), jnp.float32),
            pltpu.VMEM((LAST_ROWS, OUT), jnp.float32),
            pltpu.SemaphoreType.DMA,
            pltpu.SemaphoreType.DMA,
        ],
    )(src, dst, P1, P2, E3)


def _proj_nodes_kernel(x_ref, w12_ref, p12_ref):
    p12_ref[...] = jnp.dot(x_ref[...], w12_ref[...],
                           preferred_element_type=jnp.float32)


def _proj_edges_kernel(ea_ref, w3_ref, b_ref, e3_ref):
    e3_ref[...] = jnp.dot(ea_ref[...], w3_ref[...],
                          preferred_element_type=jnp.float32) + b_ref[...]


def _final_kernel(x_ref, agg_ref, wu_ref, b_ref, out_ref):
    xin = jnp.concatenate([x_ref[...], agg_ref[...]], axis=-1)
    out_ref[...] = jnp.dot(xin, wu_ref[...],
                           preferred_element_type=jnp.float32) + b_ref[...]


def kernel(x, edge_index, edge_attr, W_msg, b_msg, W_upd, b_upd):
    src = edge_index[0]
    dst = edge_index[1]

    W12_cat = jnp.concatenate([W_msg[:D], W_msg[D:2 * D]], axis=1)
    P12 = pl.pallas_call(
        _proj_nodes_kernel,
        out_shape=jax.ShapeDtypeStruct((N, 2 * OUT), jnp.float32),
    )(x, W12_cat)
    P1 = P12[:, :OUT]
    P2 = P12[:, OUT:]

    EB = 8000
    E3 = pl.pallas_call(
        _proj_edges_kernel,
        grid=(E // EB,),
        in_specs=[
            pl.BlockSpec((EB, DE), lambda i: (i, 0)),
            pl.BlockSpec((DE, OUT), lambda i: (0, 0)),
            pl.BlockSpec((1, OUT), lambda i: (0, 0)),
        ],
        out_specs=pl.BlockSpec((EB, OUT), lambda i: (i, 0)),
        out_shape=jax.ShapeDtypeStruct((E, OUT), jnp.float32),
    )(edge_attr, W_msg[2 * D:], b_msg.reshape(1, OUT))

    agg = _sc_agg(src, dst, P1, P2, E3)

    out = pl.pallas_call(
        _final_kernel,
        out_shape=jax.ShapeDtypeStruct((N, OUT), jnp.float32),
    )(x, agg, W_upd, b_upd.reshape(1, OUT))
    return out


# speedup vs baseline: 2.1652x; 2.1652x over previous
import jax
import jax.numpy as jnp
from jax import lax
from jax.experimental import pallas as pl
from jax.experimental.pallas import tpu as pltpu
from jax.experimental.pallas import tpu_sc as plsc

N = 10000
E = 320000
D = 128
DE = 16
OUT = 128

NC = 2
NS = 16
NW = NC * NS
RPW = 312
LAST_ROWS = N - (NW - 1) * RPW
CHUNK = 4000
NCHUNKS = E // CHUNK
G = 128
LCAP = CHUNK + G


def _sc_agg_body(src_hbm, dst_hbm, p1_hbm, e3_hbm, agg_hbm,
                 dstv, srcv, ids_buf, srcc, dstc, gsrc, gdst, gids,
                 p1b, e3b, acc, sem_lin, sem_g):
    cid = lax.axis_index("c")
    sid = lax.axis_index("s")
    wid = sid * NC + cid
    n0 = wid * RPW
    n1 = jnp.where(wid == NW - 1, N, n0 + RPW)
    lanes = lax.iota(jnp.int32, 16)

    zf = jnp.full((16,), -3e38, jnp.float32)

    def _zacc(i, carry):
        acc[i // 8, pl.ds((i % 8) * 16, 16)] = zf
        return carry

    lax.fori_loop(0, LAST_ROWS * 8, _zacc, 0)

    zi = jnp.zeros((16,), jnp.int32)

    def _zidx(i, carry):
        s = pl.ds(i * 16, 16)
        ids_buf[s] = zi
        srcc[s] = zi
        dstc[s] = zi
        return carry

    lax.fori_loop(0, LCAP // 16, _zidx, 0)

    def scan_chunk(c):
        e0 = c * CHUNK

        def scan_body(v, m):
            s = pl.ds(v * 16, 16)
            dv = dstv[s]
            sv = srcv[s]
            mask = (dv >= n0) & (dv < n1)
            ids = e0 + v * 16 + lanes
            cum = plsc.cumsum(mask.astype(jnp.int32))
            pos = m + cum - 1
            plsc.store_scatter(ids_buf, [pos], ids, mask=mask)
            plsc.store_scatter(srcc, [pos], sv, mask=mask)
            plsc.store_scatter(dstc, [pos], dv, mask=mask)
            return m + cum[15]

        return lax.fori_loop(0, CHUNK // 16, scan_body, jnp.int32(0))

    def stage_block(off):
        for k in range(G // 16):
            si = pl.ds(off + k * 16, 16)
            di = pl.ds(k * 16, 16)
            gsrc[di] = srcc[si]
            gdst[di] = dstc[si]
            gids[di] = ids_buf[si]

    def fire_gather():
        pltpu.async_copy(p1_hbm.at[gsrc], p1b, sem_g)
        pltpu.async_copy(e3_hbm.at[gids], e3b, sem_g)

    def wait_gather():
        pltpu.make_async_copy(p1_hbm.at[pl.ds(0, G)], p1b, sem_g).wait()
        pltpu.make_async_copy(p1_hbm.at[pl.ds(0, G)], e3b, sem_g).wait()

    def accumulate(m_eff):
        def acc_body(j, carry):
            base16 = (j // 16) * 16
            dvec = gdst[pl.ds(base16, 16)]
            lane = j - base16
            d = jnp.sum(jnp.where(lanes == lane, dvec, 0)) - n0
            for k in range(OUT // 16):
                sk = pl.ds(k * 16, 16)
                z = p1b[j, sk] + e3b[j, sk]
                acc[d, sk] = jnp.maximum(acc[d, sk], z)
            return carry

        lax.fori_loop(0, m_eff, acc_body, 0)

    def accumulate_extra(m_tot):
        nex = (m_tot + (G - 1)) // G - 1

        def ov(b, carry):
            off = (b + 1) * G
            stage_block(off)
            fire_gather()
            wait_gather()
            accumulate(jnp.minimum(m_tot - off, G))
            return carry

        lax.fori_loop(0, nex, ov, 0)

    pltpu.sync_copy(dst_hbm.at[pl.ds(0, CHUNK)], dstv)
    pltpu.sync_copy(src_hbm.at[pl.ds(0, CHUNK)], srcv)
    m0 = scan_chunk(0)
    stage_block(0)
    fire_gather()
    nxt0 = jnp.minimum(CHUNK, E - CHUNK)
    pltpu.async_copy(dst_hbm.at[pl.ds(nxt0, CHUNK)], dstv, sem_lin)
    pltpu.async_copy(src_hbm.at[pl.ds(nxt0, CHUNK)], srcv, sem_lin)

    def body(i, m_prev):
        early = m_prev > G

        @pl.when(early)
        def _():
            wait_gather()
            accumulate(G)
            accumulate_extra(m_prev)

        pltpu.make_async_copy(dst_hbm.at[pl.ds(0, CHUNK)], dstv,
                              sem_lin).wait()
        pltpu.make_async_copy(src_hbm.at[pl.ds(0, CHUNK)], srcv,
                              sem_lin).wait()
        m_i = scan_chunk(i)

        @pl.when(jnp.logical_not(early))
        def _():
            wait_gather()
            accumulate(jnp.minimum(m_prev, G))

        stage_block(0)
        fire_gather()
        nxt = jnp.minimum((i + 1) * CHUNK, E - CHUNK)
        pltpu.async_copy(dst_hbm.at[pl.ds(nxt, CHUNK)], dstv, sem_lin)
        pltpu.async_copy(src_hbm.at[pl.ds(nxt, CHUNK)], srcv, sem_lin)
        return m_i

    m_last = lax.fori_loop(1, NCHUNKS, body, m0)

    pltpu.make_async_copy(dst_hbm.at[pl.ds(0, CHUNK)], dstv, sem_lin).wait()
    pltpu.make_async_copy(src_hbm.at[pl.ds(0, CHUNK)], srcv, sem_lin).wait()
    wait_gather()
    accumulate(jnp.minimum(m_last, G))
    accumulate_extra(m_last)

    @pl.when(wid < NW - 1)
    def _():
        pltpu.sync_copy(acc.at[:RPW], agg_hbm.at[pl.ds(n0, RPW)])

    @pl.when(wid == NW - 1)
    def _():
        pltpu.sync_copy(acc, agg_hbm.at[pl.ds(n0, LAST_ROWS)])


def _sc_agg(src, dst, P1, E3):
    mesh = plsc.VectorSubcoreMesh(core_axis_name="c", subcore_axis_name="s")
    return pl.kernel(
        _sc_agg_body,
        out_type=jax.ShapeDtypeStruct((N, OUT), jnp.float32),
        mesh=mesh,
        compiler_params=pltpu.CompilerParams(needs_layout_passes=False),
        scratch_types=[
            pltpu.VMEM((CHUNK,), jnp.int32),
            pltpu.VMEM((CHUNK,), jnp.int32),
            pltpu.VMEM((LCAP,), jnp.int32),
            pltpu.VMEM((LCAP,), jnp.int32),
            pltpu.VMEM((LCAP,), jnp.int32),
            pltpu.VMEM((G,), jnp.int32),
            pltpu.VMEM((G,), jnp.int32),
            pltpu.VMEM((G,), jnp.int32),
            pltpu.VMEM((G, OUT), jnp.float32),
            pltpu.VMEM((G, OUT), jnp.float32),
            pltpu.VMEM((LAST_ROWS, OUT), jnp.float32),
            pltpu.SemaphoreType.DMA,
            pltpu.SemaphoreType.DMA,
        ],
    )(src, dst, P1, E3)


def _proj_nodes_kernel(x_ref, w12_ref, p12_ref):
    p12_ref[...] = jnp.dot(x_ref[...], w12_ref[...],
                           preferred_element_type=jnp.float32)


def _proj_edges_kernel(ea_ref, w3_ref, b_ref, e3_ref):
    e3_ref[...] = jnp.dot(ea_ref[...], w3_ref[...],
                          preferred_element_type=jnp.float32) + b_ref[...]


def _final_kernel(x_ref, m_ref, p2_ref, wu_ref, b_ref, out_ref):
    agg = jnp.maximum(m_ref[...] + p2_ref[...], 0.0)
    xin = jnp.concatenate([x_ref[...], agg], axis=-1)
    out_ref[...] = jnp.dot(xin, wu_ref[...],
                           preferred_element_type=jnp.float32) + b_ref[...]


def kernel(x, edge_index, edge_attr, W_msg, b_msg, W_upd, b_upd):
    src = edge_index[0]
    dst = edge_index[1]

    W12_cat = jnp.concatenate([W_msg[:D], W_msg[D:2 * D]], axis=1)
    P12 = pl.pallas_call(
        _proj_nodes_kernel,
        out_shape=jax.ShapeDtypeStruct((N, 2 * OUT), jnp.float32),
    )(x, W12_cat)
    P1 = P12[:, :OUT]
    P2 = P12[:, OUT:]

    EB = 8000
    E3 = pl.pallas_call(
        _proj_edges_kernel,
        grid=(E // EB,),
        in_specs=[
            pl.BlockSpec((EB, DE), lambda i: (i, 0)),
            pl.BlockSpec((DE, OUT), lambda i: (0, 0)),
            pl.BlockSpec((1, OUT), lambda i: (0, 0)),
        ],
        out_specs=pl.BlockSpec((EB, OUT), lambda i: (i, 0)),
        out_shape=jax.ShapeDtypeStruct((E, OUT), jnp.float32),
    )(edge_attr, W_msg[2 * D:], b_msg.reshape(1, OUT))

    m_raw = _sc_agg(src, dst, P1, E3)

    out = pl.pallas_call(
        _final_kernel,
        out_shape=jax.ShapeDtypeStruct((N, OUT), jnp.float32),
    )(x, m_raw, P2, W_upd, b_upd.reshape(1, OUT))
    return out
